# baseline (speedup 1.0000x reference)
import functools

import jax
import jax.numpy as jnp
from jax import lax
from jax.experimental import pallas as pl
from jax.experimental.pallas import tpu as pltpu
from jax.experimental.pallas import tpu_sc as plsc

N_NODES = 10000
N_PAD = 10112
D = 128
NC = 2
NS = 16
LANES = 16
B = 128
CH = 8
ROWS_PER_TILE = N_PAD // NS


def _sc_aggregate(x, src3, dst3, w3, nb):
    nchunks = nb // CH
    mesh = plsc.VectorSubcoreMesh(
        core_axis_name="c", subcore_axis_name="s", num_cores=NC, num_subcores=NS
    )

    @functools.partial(
        pl.kernel,
        out_type=jax.ShapeDtypeStruct((NC, N_PAD, D), jnp.float32),
        mesh=mesh,
        scratch_types=[
            pltpu.VMEM_SHARED((N_PAD, D), jnp.float32),
            [pltpu.VMEM((CH, B), jnp.int32)] * 2,
            [pltpu.VMEM((CH, B), jnp.int32)] * 2,
            [pltpu.VMEM((CH, B), jnp.float32)] * 2,
            [pltpu.VMEM((B, D), jnp.float32)] * 2,
            pltpu.SemaphoreType.DMA,
            [pltpu.SemaphoreType.DMA] * 2,
            [pltpu.SemaphoreType.DMA] * 2,
        ],
    )
    def k(x_hbm, src_hbm, dst_hbm, w_hbm, out_hbm,
          acc_sh, srcc, dstc, wc, rows, semi, semg, sems):
        c = lax.axis_index("c")
        s = lax.axis_index("s")
        row0 = s * ROWS_PER_TILE

        def idx_start(ci, q):
            pltpu.async_copy(src_hbm.at[s, pl.ds(ci * CH, CH)], srcc[q], semi)
            pltpu.async_copy(dst_hbm.at[s, pl.ds(ci * CH, CH)], dstc[q], semi)
            pltpu.async_copy(w_hbm.at[c, s, pl.ds(ci * CH, CH)], wc[q], semi)

        def idx_wait(ci, q):
            pltpu.make_async_copy(src_hbm.at[s, pl.ds(ci * CH, CH)], srcc[q], semi).wait()
            pltpu.make_async_copy(dst_hbm.at[s, pl.ds(ci * CH, CH)], dstc[q], semi).wait()
            pltpu.make_async_copy(w_hbm.at[c, s, pl.ds(ci * CH, CH)], wc[q], semi).wait()

        def gather_start(q, j, p):
            pltpu.async_copy(x_hbm.at[srcc[q].at[j]], rows[p], semg[p])

        def gather_wait(q, j, p):
            pltpu.make_async_copy(x_hbm.at[srcc[q].at[j]], rows[p], semg[p]).wait()

        def scatter_start(q, j, p):
            pltpu.async_copy(rows[p], acc_sh.at[dstc[q].at[j]], sems[p], add=True)

        def scatter_wait(q, j, p):
            pltpu.make_async_copy(rows[p], acc_sh.at[dstc[q].at[j]], sems[p]).wait()

        def scale(q, j, p):
            def grp(g, carry):
                wgroup = wc[q][j, pl.ds(g * LANES, LANES)]
                for jj in range(LANES):
                    wj = wgroup[jj]
                    r = g * LANES + jj
                    for kk in range(D // LANES):
                        sl = pl.ds(kk * LANES, LANES)
                        rows[p][r, sl] = rows[p][r, sl] * wj
                return carry
            lax.fori_loop(0, B // LANES, grp, 0)

        idx_start(0, 0)
        def zrow(j, carry):
            for kk in range(D // LANES):
                rows[0][j, pl.ds(kk * LANES, LANES)] = jnp.zeros((LANES,), jnp.float32)
            return carry
        lax.fori_loop(0, B, zrow, 0)
        for i in range(4):
            pltpu.sync_copy(rows[0], acc_sh.at[pl.ds(row0 + i * B, B)])
        pltpu.sync_copy(rows[0].at[pl.ds(0, ROWS_PER_TILE - 4 * B)],
                        acc_sh.at[pl.ds(row0 + 4 * B, ROWS_PER_TILE - 4 * B)])
        plsc.subcore_barrier()
        idx_wait(0, 0)
        gather_start(0, 0, 0)

        def body(t, carry):
            for ch in range(2):
                ci = t * 2 + ch
                for j in range(CH):
                    b = ci * CH + j
                    p = j % 2
                    gather_wait(ch, j, p)
                    scale(ch, j, p)
                    @pl.when(b >= 1)
                    def _():
                        jprev = (j - 1) % CH
                        chprev = ch if j >= 1 else 1 - ch
                        scatter_wait(chprev, jprev, 1 - p)
                    scatter_start(ch, j, p)
                    if j == 0:
                        @pl.when(ci + 1 < nchunks)
                        def _():
                            idx_start(ci + 1, 1 - ch)
                    if j < CH - 1:
                        gather_start(ch, j + 1, 1 - p)
                    else:
                        @pl.when(ci + 1 < nchunks)
                        def _():
                            idx_wait(ci + 1, 1 - ch)
                            gather_start(1 - ch, 0, 1 - p)
            return carry
        lax.fori_loop(0, nchunks // 2, body, 0)

        scatter_wait(1, CH - 1, 1)

        plsc.subcore_barrier()
        pltpu.sync_copy(acc_sh.at[pl.ds(row0, ROWS_PER_TILE)],
                        out_hbm.at[c, pl.ds(row0, ROWS_PER_TILE)])

    return k(x, src3, dst3, w3)


def _tc_combine(agg, W1, W2, bias):
    BM = 1000
    grid = (N_NODES // BM,)

    def body(a0, a1, w1, w2, bref, o):
        o[:, :] = (
            jnp.dot(a0[0], w1[:, :], preferred_element_type=jnp.float32)
            + jnp.dot(a1[0], w2[:, :], preferred_element_type=jnp.float32)
            + bref[:, :]
        )

    return pl.pallas_call(
        body,
        grid=grid,
        in_specs=[
            pl.BlockSpec((1, BM, D), lambda i: (0, i, 0)),
            pl.BlockSpec((1, BM, D), lambda i: (1, i, 0)),
            pl.BlockSpec((D, D), lambda i: (0, 0)),
            pl.BlockSpec((D, D), lambda i: (0, 0)),
            pl.BlockSpec((1, D), lambda i: (0, 0)),
        ],
        out_specs=pl.BlockSpec((BM, D), lambda i: (i, 0)),
        out_shape=jax.ShapeDtypeStruct((N_NODES, D), jnp.float32),
    )(agg, agg, W1, W2, bias)


def kernel(x, edge_index, edge_weight_src_to_tgt, edge_weight_tgt_to_src,
           W_src_to_dst, W_dst_to_src, b_src_to_dst, b_dst_to_src):
    E = edge_index.shape[1]
    gran = NS * B * CH * 2
    epad = -(-E // gran) * gran
    nb = epad // (NS * B)
    pad = epad - E
    src = jnp.pad(edge_index[0], (0, pad)).reshape(NS, nb, B)
    dst = jnp.pad(edge_index[1], (0, pad)).reshape(NS, nb, B)
    w1 = jnp.pad(edge_weight_src_to_tgt[:, 0], (0, pad))
    w2 = jnp.pad(edge_weight_tgt_to_src[:, 0], (0, pad))
    w = jnp.stack([w1, w2]).reshape(NC, NS, nb, B)
    agg = _sc_aggregate(x, src, dst, w, nb)
    bias = (b_src_to_dst + b_dst_to_src).reshape(1, D)
    return _tc_combine(agg, W_src_to_dst, W_dst_to_src, bias)


# speedup vs baseline: 1.0555x; 1.0555x over previous
import functools

import jax
import jax.numpy as jnp
from jax import lax
from jax.experimental import pallas as pl
from jax.experimental.pallas import tpu as pltpu
from jax.experimental.pallas import tpu_sc as plsc

N_NODES = 10000
N_PAD = 10112
D = 128
NC = 2
NS = 16
LANES = 16
B = 128
CH = 4
ROWS_PER_TILE = N_PAD // NS


def _sc_aggregate(x, src3, dst3, w3, nb):
    nchunks = nb // CH
    mesh = plsc.VectorSubcoreMesh(
        core_axis_name="c", subcore_axis_name="s", num_cores=NC, num_subcores=NS
    )

    @functools.partial(
        pl.kernel,
        out_type=jax.ShapeDtypeStruct((NC, N_PAD, D), jnp.float32),
        mesh=mesh,
        scratch_types=[
            pltpu.VMEM_SHARED((N_PAD, D), jnp.float32),
            [pltpu.VMEM((CH, B), jnp.int32)] * 2,
            [pltpu.VMEM((CH, B), jnp.int32)] * 2,
            [pltpu.VMEM((CH, B), jnp.float32)] * 2,
            [pltpu.VMEM((B, D), jnp.float32)] * 2,
            pltpu.SemaphoreType.DMA,
            [pltpu.SemaphoreType.DMA] * 2,
            [pltpu.SemaphoreType.DMA] * 2,
        ],
    )
    def k(x_hbm, src_hbm, dst_hbm, w_hbm, out_hbm,
          acc_sh, srcc, dstc, wc, rows, semi, semg, sems):
        c = lax.axis_index("c")
        s = lax.axis_index("s")
        row0 = s * ROWS_PER_TILE

        def idx_start(ci, q):
            pltpu.async_copy(src_hbm.at[s, pl.ds(ci * CH, CH)], srcc[q], semi)
            pltpu.async_copy(dst_hbm.at[s, pl.ds(ci * CH, CH)], dstc[q], semi)
            pltpu.async_copy(w_hbm.at[c, s, pl.ds(ci * CH, CH)], wc[q], semi)

        def idx_wait(ci, q):
            pltpu.make_async_copy(src_hbm.at[s, pl.ds(ci * CH, CH)], srcc[q], semi).wait()
            pltpu.make_async_copy(dst_hbm.at[s, pl.ds(ci * CH, CH)], dstc[q], semi).wait()
            pltpu.make_async_copy(w_hbm.at[c, s, pl.ds(ci * CH, CH)], wc[q], semi).wait()

        def gather_start(q, j, p):
            pltpu.async_copy(x_hbm.at[srcc[q].at[j]], rows[p], semg[p])

        def gather_wait(q, j, p):
            pltpu.make_async_copy(x_hbm.at[srcc[q].at[j]], rows[p], semg[p]).wait()

        def scatter_start(q, j, p):
            pltpu.async_copy(rows[p], acc_sh.at[dstc[q].at[j]], sems[p], add=True)

        def scatter_wait(q, j, p):
            pltpu.make_async_copy(rows[p], acc_sh.at[dstc[q].at[j]], sems[p]).wait()

        def scale(q, j, p):
            @plsc.parallel_loop(0, B // LANES)
            def grp(g):
                wgroup = wc[q][j, pl.ds(g * LANES, LANES)]
                for jj in range(LANES):
                    wj = wgroup[jj]
                    r = g * LANES + jj
                    for kk in range(D // LANES):
                        sl = pl.ds(kk * LANES, LANES)
                        rows[p][r, sl] = rows[p][r, sl] * wj

        idx_start(0, 0)
        def zrow(j, carry):
            for kk in range(D // LANES):
                rows[0][j, pl.ds(kk * LANES, LANES)] = jnp.zeros((LANES,), jnp.float32)
            return carry
        lax.fori_loop(0, B, zrow, 0)
        for i in range(4):
            pltpu.sync_copy(rows[0], acc_sh.at[pl.ds(row0 + i * B, B)])
        pltpu.sync_copy(rows[0].at[pl.ds(0, ROWS_PER_TILE - 4 * B)],
                        acc_sh.at[pl.ds(row0 + 4 * B, ROWS_PER_TILE - 4 * B)])
        plsc.subcore_barrier()
        idx_wait(0, 0)
        gather_start(0, 0, 0)

        def body(t, carry):
            for ch in range(2):
                ci = t * 2 + ch
                for j in range(CH):
                    b = ci * CH + j
                    p = j % 2
                    gather_wait(ch, j, p)
                    @pl.when(b >= 1)
                    def _():
                        jprev = (j - 1) % CH
                        chprev = ch if j >= 1 else 1 - ch
                        scatter_wait(chprev, jprev, 1 - p)
                    if j == 0:
                        @pl.when(ci + 1 < nchunks)
                        def _():
                            idx_start(ci + 1, 1 - ch)
                    if j < CH - 1:
                        gather_start(ch, j + 1, 1 - p)
                    else:
                        @pl.when(ci + 1 < nchunks)
                        def _():
                            idx_wait(ci + 1, 1 - ch)
                            gather_start(1 - ch, 0, 1 - p)
                    scale(ch, j, p)
                    scatter_start(ch, j, p)
            return carry
        lax.fori_loop(0, nchunks // 2, body, 0)

        scatter_wait(1, CH - 1, 1)

        plsc.subcore_barrier()
        pltpu.sync_copy(acc_sh.at[pl.ds(row0, ROWS_PER_TILE)],
                        out_hbm.at[c, pl.ds(row0, ROWS_PER_TILE)])

    return k(x, src3, dst3, w3)


def _tc_combine(agg, W1, W2, bias):
    BM = 1000
    grid = (N_NODES // BM,)

    def body(a0, a1, w1, w2, bref, o):
        o[:, :] = (
            jnp.dot(a0[0], w1[:, :], preferred_element_type=jnp.float32)
            + jnp.dot(a1[0], w2[:, :], preferred_element_type=jnp.float32)
            + bref[:, :]
        )

    return pl.pallas_call(
        body,
        grid=grid,
        in_specs=[
            pl.BlockSpec((1, BM, D), lambda i: (0, i, 0)),
            pl.BlockSpec((1, BM, D), lambda i: (1, i, 0)),
            pl.BlockSpec((D, D), lambda i: (0, 0)),
            pl.BlockSpec((D, D), lambda i: (0, 0)),
            pl.BlockSpec((1, D), lambda i: (0, 0)),
        ],
        out_specs=pl.BlockSpec((BM, D), lambda i: (i, 0)),
        out_shape=jax.ShapeDtypeStruct((N_NODES, D), jnp.float32),
    )(agg, agg, W1, W2, bias)


def kernel(x, edge_index, edge_weight_src_to_tgt, edge_weight_tgt_to_src,
           W_src_to_dst, W_dst_to_src, b_src_to_dst, b_dst_to_src):
    E = edge_index.shape[1]
    gran = NS * B * CH * 2
    epad = -(-E // gran) * gran
    nb = epad // (NS * B)
    pad = epad - E
    src = jnp.pad(edge_index[0], (0, pad)).reshape(NS, nb, B)
    dst = jnp.pad(edge_index[1], (0, pad)).reshape(NS, nb, B)
    w1 = jnp.pad(edge_weight_src_to_tgt[:, 0], (0, pad))
    w2 = jnp.pad(edge_weight_tgt_to_src[:, 0], (0, pad))
    w = jnp.stack([w1, w2]).reshape(NC, NS, nb, B)
    agg = _sc_aggregate(x, src, dst, w, nb)
    bias = (b_src_to_dst + b_dst_to_src).reshape(1, D)
    return _tc_combine(agg, W_src_to_dst, W_dst_to_src, bias)
